# baseline (speedup 1.0000x reference)
import functools

import jax
import jax.numpy as jnp
from jax import lax
from jax.experimental import pallas as pl
from jax.experimental.pallas import tpu as pltpu
from jax.experimental.pallas import tpu_sc as plsc

N = 50000
D = 256
B = 512
G = 8
H = 128
S = B * G
NC = 2
NS = 16
NW = NC * NS
BPT = B // NW
RPT = BPT * G
K = 80
STRIDE = K - 8
JUNK = RPT


def _sc_segment_sum(node_feature, combined_ids):
    mesh = plsc.VectorSubcoreMesh(
        core_axis_name="c", subcore_axis_name="s",
        num_cores=NC, num_subcores=NS)

    @functools.partial(
        pl.kernel,
        mesh=mesh,
        out_type=jax.ShapeDtypeStruct((S, D), jnp.float32),
        scratch_types=[
            pltpu.VMEM((N,), jnp.int32),
            pltpu.VMEM((RPT + 1, D), jnp.float32),
            pltpu.VMEM((K, D), jnp.float32),
            pltpu.VMEM((K, D), jnp.float32),
            pltpu.SemaphoreType.DMA,
            pltpu.SemaphoreType.DMA,
        ],
    )
    def seg_sum(nodes, cmb, out, ids_full, acc, rows0, rows1, sem0, sem1):
        cid = lax.axis_index("c")
        sid = lax.axis_index("s")
        wid = cid * NS + sid
        s_first = wid * RPT

        ids_cp = pltpu.async_copy(cmb, ids_full, sem0)
        zv = jnp.zeros((16,), jnp.float32)

        @pl.loop(0, RPT + 1)
        def _(r):
            for c in range(D // 16):
                acc[r, pl.ds(c * 16, 16)] = zv

        ids_cp.wait()

        nwin = N // 16

        def lower_bound(thr):
            b = jnp.int32(0)
            sz = 4096
            while sz >= 1:
                nb = b + sz
                widx = jnp.minimum(nb - 1, nwin - 1)
                v = ids_full[pl.ds(widx * 16, 16)]
                b = jnp.where((nb <= nwin) & (v[15] < thr), nb, b)
                sz //= 2
            widx = jnp.minimum(b, nwin - 1)
            v = ids_full[pl.ds(widx * 16, 16)]
            cnt = jnp.int32(0)
            for l in range(16):
                cnt = cnt + jnp.where(v[l] < thr, 1, 0)
            return b * 16 + jnp.where(b < nwin, cnt, 0)

        lo = lower_bound(s_first)
        hi = lower_bound(s_first + RPT)

        nblk = (hi - lo + (STRIDE - 1)) // STRIDE

        def blk_base(j):
            pos = lo + j * STRIDE
            return pos, pl.multiple_of(
                jnp.minimum(pos - (pos % 8), N - K), 8)

        def accumulate(rows, base, pos, lim):
            @pl.loop(0, K // 16)
            def _(t):
                i0 = t * 16
                ids16 = ids_full[pl.ds(base + i0, 16)] - s_first
                for l in range(16):
                    g = base + i0 + l
                    valid = (g >= pos) & (g < lim)
                    row = jnp.where(valid, ids16[l], JUNK)
                    vals = [rows[i0 + l, pl.ds(c * 16, 16)]
                            for c in range(D // 16)]
                    for c in range(D // 16):
                        plsc.addupdate(acc.at[row, pl.ds(c * 16, 16)],
                                       vals[c])

        bufs = ((rows0, sem0), (rows1, sem1))

        @pl.when(nblk > 0)
        def _():
            _, base0 = blk_base(0)
            pltpu.async_copy(nodes.at[pl.ds(base0, K)], rows0, sem0)

        @pl.loop(0, nblk)
        def _(j):
            pos, base = blk_base(j)
            lim = jnp.minimum(pos + STRIDE, hi)

            def run_parity(b):
                rows, sem = bufs[b]
                nrows, nsem = bufs[1 - b]

                @pl.when(j % 2 == b)
                def _():
                    pltpu.make_async_copy(
                        nodes.at[pl.ds(base, K)], rows, sem).wait()

                    @pl.when(j + 1 < nblk)
                    def _():
                        _, nbase = blk_base(j + 1)
                        pltpu.async_copy(
                            nodes.at[pl.ds(nbase, K)], nrows, nsem)

                    accumulate(rows, base, pos, lim)

            run_parity(0)
            run_parity(1)

        pltpu.sync_copy(acc.at[pl.ds(0, RPT)], out.at[pl.ds(wid * RPT, RPT)])

    return seg_sum(node_feature, combined_ids)


def _tc_mlp_body(seg_ref, sqT_ref, wW1_ref, wb1_ref, wW2_ref, wb2_ref,
                 vW1_ref, vb1_ref, vW2_ref, vb2_ref, out_ref):
    seg = seg_ref[...]
    h = jnp.maximum(
        jnp.dot(seg, wW1_ref[...], preferred_element_type=jnp.float32)
        + wb1_ref[...], 0.0)
    wo = jnp.dot(h, wW2_ref[...], preferred_element_type=jnp.float32)
    wo = jnp.abs(wo + wb2_ref[...]).reshape(B, G)
    q_tot = jnp.sum(wo * sqT_ref[...], axis=1)
    v_in = seg.reshape(B, G, D).sum(axis=1)
    hv = jnp.maximum(
        jnp.dot(v_in, vW1_ref[...], preferred_element_type=jnp.float32)
        + vb1_ref[...], 0.0)
    v = jnp.dot(hv, vW2_ref[...], preferred_element_type=jnp.float32)
    v = (v + vb2_ref[...]).reshape(B)
    out_ref[0, :] = q_tot + v


def kernel(node_feature, batch_seg, assignment, sub_q_tots,
           w_W1, w_b1, w_W2, w_b2, v_W1, v_b1, v_W2, v_b2):
    combined = (batch_seg.astype(jnp.int32) * G
                + assignment.astype(jnp.int32))
    seg = _sc_segment_sum(node_feature, combined)

    out = pl.pallas_call(
        _tc_mlp_body,
        out_shape=jax.ShapeDtypeStruct((1, B), jnp.float32),
    )(seg, sub_q_tots.T,
      w_W1, w_b1.reshape(1, -1), w_W2, w_b2.reshape(1, 1),
      v_W1, v_b1.reshape(1, -1), v_W2, v_b2.reshape(1, 1))
    return out.reshape(B)


# speedup vs baseline: 1.0512x; 1.0512x over previous
import functools

import jax
import jax.numpy as jnp
from jax import lax
from jax.experimental import pallas as pl
from jax.experimental.pallas import tpu as pltpu
from jax.experimental.pallas import tpu_sc as plsc

N = 50000
D = 256
B = 512
G = 8
H = 128
S = B * G
NC = 2
NS = 16
NW = NC * NS
BPT = B // NW
RPT = BPT * G
K = 80
JUNK = RPT


def _sc_segment_sum(node_feature, combined_ids):
    mesh = plsc.VectorSubcoreMesh(
        core_axis_name="c", subcore_axis_name="s",
        num_cores=NC, num_subcores=NS)

    @functools.partial(
        pl.kernel,
        mesh=mesh,
        out_type=jax.ShapeDtypeStruct((S, D), jnp.float32),
        scratch_types=[
            pltpu.VMEM((N,), jnp.int32),
            pltpu.VMEM((RPT + 1, D), jnp.float32),
            pltpu.VMEM((K, D), jnp.float32),
            pltpu.VMEM((K, D), jnp.float32),
            pltpu.SemaphoreType.DMA,
            pltpu.SemaphoreType.DMA,
        ],
    )
    def seg_sum(nodes, cmb, out, ids_full, acc, rows0, rows1, sem0, sem1):
        cid = lax.axis_index("c")
        sid = lax.axis_index("s")
        wid = cid * NS + sid
        s_first = wid * RPT

        ids_cp = pltpu.async_copy(cmb, ids_full, sem0)
        zv = jnp.zeros((16,), jnp.float32)

        @pl.loop(0, RPT + 1)
        def _(r):
            for c in range(D // 16):
                acc[r, pl.ds(c * 16, 16)] = zv

        ids_cp.wait()

        nwin = N // 16

        def lower_bound(thr):
            b = jnp.int32(0)
            sz = 4096
            while sz >= 1:
                nb = b + sz
                widx = jnp.minimum(nb - 1, nwin - 1)
                v = ids_full[pl.ds(widx * 16, 16)]
                b = jnp.where((nb <= nwin) & (v[15] < thr), nb, b)
                sz //= 2
            widx = jnp.minimum(b, nwin - 1)
            v = ids_full[pl.ds(widx * 16, 16)]
            cnt = jnp.int32(0)
            for l in range(16):
                cnt = cnt + jnp.where(v[l] < thr, 1, 0)
            return b * 16 + jnp.where(b < nwin, cnt, 0)

        lo = lower_bound(s_first)
        hi = lower_bound(s_first + RPT)

        start = pl.multiple_of(lo - (lo % 8), 8)
        nblk = (hi - start + (K - 1)) // K

        def blk_base(j):
            pos = start + j * K
            return pos, pl.multiple_of(jnp.minimum(pos, N - K), 8)

        def accumulate(rows, base, lower, lim):
            @pl.loop(0, K // 16)
            def _(t):
                i0 = t * 16
                ids16 = ids_full[pl.ds(base + i0, 16)] - s_first
                for l in range(16):
                    g = base + i0 + l
                    valid = (g >= lower) & (g < lim)
                    row = jnp.where(valid, ids16[l], JUNK)
                    vals = [rows[i0 + l, pl.ds(c * 16, 16)]
                            for c in range(D // 16)]
                    for c in range(D // 16):
                        plsc.addupdate(acc.at[row, pl.ds(c * 16, 16)],
                                       vals[c])

        bufs = ((rows0, sem0), (rows1, sem1))

        @pl.when(nblk > 0)
        def _():
            _, base0 = blk_base(0)
            pltpu.async_copy(nodes.at[pl.ds(base0, K)], rows0, sem0)

        @pl.loop(0, nblk)
        def _(j):
            pos, base = blk_base(j)
            lower = jnp.maximum(lo, pos)

            def run_parity(b):
                rows, sem = bufs[b]
                nrows, nsem = bufs[1 - b]

                @pl.when(j % 2 == b)
                def _():
                    pltpu.make_async_copy(
                        nodes.at[pl.ds(base, K)], rows, sem).wait()

                    @pl.when(j + 1 < nblk)
                    def _():
                        _, nbase = blk_base(j + 1)
                        pltpu.async_copy(
                            nodes.at[pl.ds(nbase, K)], nrows, nsem)

                    accumulate(rows, base, lower, hi)

            run_parity(0)
            run_parity(1)

        pltpu.sync_copy(acc.at[pl.ds(0, RPT)], out.at[pl.ds(wid * RPT, RPT)])

    return seg_sum(node_feature, combined_ids)


def _tc_mlp_body(seg_ref, sqT_ref, wW1_ref, wb1_ref, wW2_ref, wb2_ref,
                 vW1_ref, vb1_ref, vW2_ref, vb2_ref, out_ref):
    seg = seg_ref[...]
    h = jnp.maximum(
        jnp.dot(seg, wW1_ref[...], preferred_element_type=jnp.float32)
        + wb1_ref[...], 0.0)
    wo = jnp.dot(h, wW2_ref[...], preferred_element_type=jnp.float32)
    wo = jnp.abs(wo + wb2_ref[...]).reshape(B, G)
    q_tot = jnp.sum(wo * sqT_ref[...], axis=1)
    v_in = seg.reshape(B, G, D).sum(axis=1)
    hv = jnp.maximum(
        jnp.dot(v_in, vW1_ref[...], preferred_element_type=jnp.float32)
        + vb1_ref[...], 0.0)
    v = jnp.dot(hv, vW2_ref[...], preferred_element_type=jnp.float32)
    v = (v + vb2_ref[...]).reshape(B)
    out_ref[0, :] = q_tot + v


def kernel(node_feature, batch_seg, assignment, sub_q_tots,
           w_W1, w_b1, w_W2, w_b2, v_W1, v_b1, v_W2, v_b2):
    combined = (batch_seg.astype(jnp.int32) * G
                + assignment.astype(jnp.int32))
    seg = _sc_segment_sum(node_feature, combined)

    out = pl.pallas_call(
        _tc_mlp_body,
        out_shape=jax.ShapeDtypeStruct((1, B), jnp.float32),
    )(seg, sub_q_tots.T,
      w_W1, w_b1.reshape(1, -1), w_W2, w_b2.reshape(1, 1),
      v_W1, v_b1.reshape(1, -1), v_W2, v_b2.reshape(1, 1))
    return out.reshape(B)


# speedup vs baseline: 1.0521x; 1.0009x over previous
import functools

import jax
import jax.numpy as jnp
from jax import lax
from jax.experimental import pallas as pl
from jax.experimental.pallas import tpu as pltpu
from jax.experimental.pallas import tpu_sc as plsc

N = 50000
D = 256
B = 512
G = 8
H = 128
S = B * G
NC = 2
NS = 16
NW = NC * NS
BPT = B // NW
RPT = BPT * G
K = 80
JUNK = RPT


def _sc_segment_sum(node_feature, combined_ids):
    mesh = plsc.VectorSubcoreMesh(
        core_axis_name="c", subcore_axis_name="s",
        num_cores=NC, num_subcores=NS)

    @functools.partial(
        pl.kernel,
        mesh=mesh,
        out_type=jax.ShapeDtypeStruct((S, D), jnp.float32),
        scratch_types=[
            pltpu.VMEM((N,), jnp.int32),
            pltpu.VMEM((RPT + 1, D), jnp.float32),
            pltpu.VMEM((K, D), jnp.float32),
            pltpu.VMEM((K, D), jnp.float32),
            pltpu.SemaphoreType.DMA,
            pltpu.SemaphoreType.DMA,
        ],
    )
    def seg_sum(nodes, cmb, out, ids_full, acc, rows0, rows1, sem0, sem1):
        cid = lax.axis_index("c")
        sid = lax.axis_index("s")
        wid = cid * NS + sid
        s_first = wid * RPT

        ids_cp = pltpu.async_copy(cmb, ids_full, sem0)
        zv = jnp.zeros((16,), jnp.float32)

        @pl.loop(0, RPT + 1)
        def _(r):
            for c in range(D // 16):
                acc[r, pl.ds(c * 16, 16)] = zv

        ids_cp.wait()

        nwin = N // 16

        def lower_bound(thr):
            b = jnp.int32(0)
            sz = 4096
            while sz >= 1:
                nb = b + sz
                widx = jnp.minimum(nb - 1, nwin - 1)
                v = ids_full[pl.ds(widx * 16, 16)]
                b = jnp.where((nb <= nwin) & (v[15] < thr), nb, b)
                sz //= 2
            widx = jnp.minimum(b, nwin - 1)
            v = ids_full[pl.ds(widx * 16, 16)]
            cnt = jnp.int32(0)
            for l in range(16):
                cnt = cnt + jnp.where(v[l] < thr, 1, 0)
            return b * 16 + jnp.where(b < nwin, cnt, 0)

        lo = lower_bound(s_first)
        hi = lower_bound(s_first + RPT)

        start = pl.multiple_of(lo - (lo % 8), 8)
        nblk = (hi - start + (K - 1)) // K

        def blk_base(j):
            pos = start + j * K
            return pos, pl.multiple_of(jnp.minimum(pos, N - K), 8)

        def accumulate(rows, base, lower, lim):
            @pl.loop(0, K // 16)
            def _(t):
                i0 = t * 16
                ids16 = ids_full[pl.ds(base + i0, 16)] - s_first
                for l in range(16):
                    g = base + i0 + l
                    valid = (g >= lower) & (g < lim)
                    row = jnp.where(valid, ids16[l], JUNK)
                    vals = [rows[i0 + l, pl.ds(c * 16, 16)]
                            for c in range(D // 16)]
                    for c in range(D // 16):
                        plsc.addupdate(acc.at[row, pl.ds(c * 16, 16)],
                                       vals[c])

        bufs = ((rows0, sem0), (rows1, sem1))

        @pl.when(nblk > 0)
        def _():
            _, base0 = blk_base(0)
            pltpu.async_copy(nodes.at[pl.ds(base0, K)], rows0, sem0)

        @pl.loop(0, nblk)
        def _(j):
            pos, base = blk_base(j)
            lower = jnp.maximum(lo, pos)

            def run_parity(b):
                rows, sem = bufs[b]
                nrows, nsem = bufs[1 - b]

                @pl.when(j % 2 == b)
                def _():
                    pltpu.make_async_copy(
                        nodes.at[pl.ds(base, K)], rows, sem).wait()

                    @pl.when(j + 1 < nblk)
                    def _():
                        _, nbase = blk_base(j + 1)
                        pltpu.async_copy(
                            nodes.at[pl.ds(nbase, K)], nrows, nsem)

                    accumulate(rows, base, lower, hi)

            run_parity(0)
            run_parity(1)

        pltpu.sync_copy(acc.at[pl.ds(0, RPT)], out.at[pl.ds(wid * RPT, RPT)])

    return seg_sum(node_feature, combined_ids)


def _tc_mlp_body(seg_ref, sqT_ref, wW1_ref, wb1_ref, wW2_ref, wb2_ref,
                 vW1_ref, vb1_ref, vW2_ref, vb2_ref, out_ref):
    seg = seg_ref[...]
    h = jnp.maximum(
        jnp.dot(seg, wW1_ref[...], preferred_element_type=jnp.float32)
        + wb1_ref[...], 0.0)
    wo = jnp.dot(h, wW2_ref[...], preferred_element_type=jnp.float32)
    wo = jnp.abs(wo + wb2_ref[...]).reshape(B, G)
    q_tot = jnp.sum(wo * sqT_ref[...].T, axis=1)
    v_in = seg.reshape(B, G, D).sum(axis=1)
    hv = jnp.maximum(
        jnp.dot(v_in, vW1_ref[...], preferred_element_type=jnp.float32)
        + vb1_ref[...], 0.0)
    v = jnp.dot(hv, vW2_ref[...], preferred_element_type=jnp.float32)
    v = (v + vb2_ref[...]).reshape(B)
    out_ref[0, :] = q_tot + v


def kernel(node_feature, batch_seg, assignment, sub_q_tots,
           w_W1, w_b1, w_W2, w_b2, v_W1, v_b1, v_W2, v_b2):
    combined = (batch_seg.astype(jnp.int32) * G
                + assignment.astype(jnp.int32))
    seg = _sc_segment_sum(node_feature, combined)

    out = pl.pallas_call(
        _tc_mlp_body,
        out_shape=jax.ShapeDtypeStruct((1, B), jnp.float32),
    )(seg, sub_q_tots,
      w_W1, w_b1.reshape(1, -1), w_W2, w_b2.reshape(1, 1),
      v_W1, v_b1.reshape(1, -1), v_W2, v_b2.reshape(1, 1))
    return out.reshape(B)
